# baseline (speedup 1.0000x reference)
import functools

import jax
import jax.numpy as jnp
from jax import lax
from jax.experimental import pallas as pl
from jax.experimental.pallas import tpu as pltpu

_C = 21
_O = 16
_LANES = 128
_R = 69
_PP = _R * _LANES
_IPS = 2


def _rmax(x):
    return jnp.max(jnp.max(x, axis=0, keepdims=True), axis=1, keepdims=True)


def _rmin(x):
    return jnp.min(jnp.min(x, axis=0, keepdims=True), axis=1, keepdims=True)


def _rsum(x):
    return jnp.sum(jnp.sum(x, axis=0, keepdims=True), axis=1, keepdims=True)


def _rsum12(x):
    return jnp.sum(jnp.sum(x, axis=2, keepdims=True), axis=1, keepdims=True)


def _rmax12(x):
    return jnp.max(jnp.max(x, axis=2, keepdims=True), axis=1, keepdims=True)


def _smooth_l1(x):
    ax = jnp.abs(x)
    return jnp.where(ax < 1.0, 0.5 * ax * ax, ax - 0.5)


def _one_image(P, targ_ref, cs_ref, ls_ref, pri_ref, j):
    lin = (lax.broadcasted_iota(jnp.int32, (_R, _LANES), 0) * _LANES
           + lax.broadcasted_iota(jnp.int32, (_R, _LANES), 1))
    valid = lin < P

    cx = pri_ref[0]
    cy = pri_ref[1]
    pw = pri_ref[2]
    ph = pri_ref[3]
    px1 = cx - pw / 2.0
    py1 = cy - ph / 2.0
    px2 = cx + pw / 2.0
    py2 = cy + ph / 2.0
    area_p = (px2 - px1) * (py2 - py1)

    tx1 = [targ_ref[j, o, 0] for o in range(_O)]
    ty1 = [targ_ref[j, o, 1] for o in range(_O)]
    tx2 = [targ_ref[j, o, 2] for o in range(_O)]
    ty2 = [targ_ref[j, o, 3] for o in range(_O)]
    tlab = [targ_ref[j, o, 4] for o in range(_O)]

    bt_over = jnp.full((_R, _LANES), -1.0, jnp.float32)
    bt_idx = jnp.zeros((_R, _LANES), jnp.int32)
    bp = []
    for o in range(_O):
        iw = jnp.clip(jnp.minimum(tx2[o], px2) - jnp.maximum(tx1[o], px1),
                      0.0, None)
        ih = jnp.clip(jnp.minimum(ty2[o], py2) - jnp.maximum(ty1[o], py1),
                      0.0, None)
        inter = iw * ih
        area_t = (tx2[o] - tx1[o]) * (ty2[o] - ty1[o])
        iou = inter / (area_t + area_p - inter + 1e-10)
        upd = iou > bt_over
        bt_idx = jnp.where(upd, o, bt_idx)
        bt_over = jnp.where(upd, iou, bt_over)
        m = _rmax(iou)
        bp.append(_rmin(jnp.where(iou == m, lin, jnp.int32(2 ** 30))))
    for o in range(_O):
        mo = lin == bp[o]
        bt_over = jnp.where(mo, 2.0, bt_over)
        bt_idx = jnp.where(mo, o, bt_idx)

    mx1 = jnp.zeros_like(cx)
    my1 = jnp.zeros_like(cx)
    mx2 = jnp.zeros_like(cx)
    my2 = jnp.zeros_like(cx)
    lab = jnp.zeros_like(cx)
    for o in range(_O):
        sel = bt_idx == o
        mx1 = jnp.where(sel, tx1[o], mx1)
        my1 = jnp.where(sel, ty1[o], my1)
        mx2 = jnp.where(sel, tx2[o], mx2)
        my2 = jnp.where(sel, ty2[o], my2)
        lab = jnp.where(sel, tlab[o], lab)
    conf_t = jnp.where(bt_over < 0.5, 0.0, lab + 1.0)
    pos = conf_t > 0.0

    ecx = ((mx1 + mx2) / 2.0 - cx) / (0.1 * pw)
    ecy = ((my1 + my2) / 2.0 - cy) / (0.1 * ph)
    ew = jnp.log(jnp.clip((mx2 - mx1) / pw, 1e-10, None)) / 0.2
    eh = jnp.log(jnp.clip((my2 - my1) / ph, 1e-10, None)) / 0.2

    lx = ls_ref[4 * j + 0]
    ly = ls_ref[4 * j + 1]
    lw = ls_ref[4 * j + 2]
    lh = ls_ref[4 * j + 3]
    dcx = cx + lx * 0.1 * pw
    dcy = cy + ly * 0.1 * ph
    dw = pw * jnp.exp(lw * 0.2)
    dh = ph * jnp.exp(lh * 0.2)

    sl = (_smooth_l1((dcx - dw / 2.0) - ecx)
          + _smooth_l1((dcy - dh / 2.0) - ecy)
          + _smooth_l1((dcx + dw / 2.0) - ew)
          + _smooth_l1((dcy + dh / 2.0) - eh))
    loss_l_img = _rsum(jnp.where(pos, sl, 0.0))

    m0 = cs_ref[_C * j + 0]
    for c in range(1, _C):
        m0 = jnp.maximum(m0, cs_ref[_C * j + c])
    c0 = cs_ref[_C * j + 0]
    s = jnp.exp(c0 - m0)
    g = c0
    for c in range(1, _C):
        cc = cs_ref[_C * j + c]
        s = s + jnp.exp(cc - m0)
        g = jnp.where(conf_t == c, cc, g)
    lse = jnp.log(s) + m0
    ce = lse - g
    lca = jnp.where(jnp.logical_and(valid, jnp.logical_not(pos)), ce, 0.0)

    npos = _rsum(jnp.where(pos, 1, 0))
    pos_ce = _rsum(jnp.where(pos, ce, 0.0))
    return loss_l_img, pos_ce, npos, lax.bitcast_convert_type(lca, jnp.int32)


def _mb_kernel(P, B, targ_ref, conf_ref, loc_ref, pri_ref, ll_ref, lc_ref,
               acc_ref, bits_ref, npk_ref, cs_ref, ls_ref):
    b = pl.program_id(0)
    nsteps = B // _IPS

    @pl.when(b == 0)
    def _init():
        acc_ref[0] = 0.0
        acc_ref[1] = 0.0
        acc_ref[2] = 0.0

    @pl.when(b < nsteps)
    def _per_step():
        ident = (lax.broadcasted_iota(jnp.int32, (_LANES, _LANES), 0)
                 == lax.broadcasted_iota(jnp.int32, (_LANES, _LANES), 1)
                 ).astype(jnp.float32)
        dn = (((0,), (0,)), ((), ()))
        for j in range(_IPS):
            for r in range(_R):
                rows = _LANES if (r + 1) * _LANES <= P else P - r * _LANES
                idr = ident if rows == _LANES else ident[:rows]
                xc = conf_ref[j, pl.ds(r * _LANES, rows), :]
                cs_ref[pl.ds(j * _C, _C), r, :] = lax.dot_general(
                    xc, idr, dn, preferred_element_type=jnp.float32)
                xl = loc_ref[j, pl.ds(r * _LANES, rows), :]
                ls_ref[pl.ds(j * 4, 4), r, :] = lax.dot_general(
                    xl, idr, dn, preferred_element_type=jnp.float32)
        for j in range(_IPS):
            loss_l_img, pos_ce, npos, bits = _one_image(
                P, targ_ref, cs_ref, ls_ref, pri_ref, j)
            img = b * _IPS + j
            bits_ref[img] = bits
            npk_ref[pl.ds(img, 1)] = npos
            acc_ref[0] = acc_ref[0] + loss_l_img[0, 0]
            acc_ref[1] = acc_ref[1] + pos_ce[0, 0]
            acc_ref[2] = acc_ref[2] + npos[0, 0].astype(jnp.float32)

    @pl.when(b == nsteps)
    def _mine():
        bits3 = bits_ref[...]
        lca3 = lax.bitcast_convert_type(bits3, jnp.float32)
        k3 = jnp.minimum(3 * npk_ref[...], P - 1).reshape(B, 1, 1)

        def bs_body(_, carry):
            lo, hi = carry
            q = jnp.maximum((hi - lo) // 4, 1)
            m1 = lo + q
            m2 = lo + 2 * q
            m3 = lo + 3 * q
            c1 = _rsum12(jnp.where(bits3 >= m1, 1, 0)) >= k3
            c2 = _rsum12(jnp.where(bits3 >= m2, 1, 0)) >= k3
            c3 = _rsum12(jnp.where(bits3 >= m3, 1, 0)) >= k3
            nlo = jnp.where(c3, m3, jnp.where(c2, m2, jnp.where(c1, m1, lo)))
            nhi = jnp.where(jnp.logical_not(c1), m1,
                            jnp.where(jnp.logical_not(c2), m2,
                                      jnp.where(jnp.logical_not(c3), m3, hi)))
            return nlo, nhi

        lo0 = jnp.zeros((B, 1, 1), jnp.int32)
        hi0 = jnp.full((B, 1, 1), 0x7F800000, jnp.int32)
        lo, _ = lax.fori_loop(0, 17, bs_body, (lo0, hi0))

        gt = bits3 > lo
        cnt_gt = _rsum12(jnp.where(gt, 1, 0))
        sum_gt = _rsum12(jnp.where(gt, lca3, 0.0))
        tval = _rmax12(jnp.where(bits3 <= lo, lca3, 0.0))
        neg = sum_gt + (k3 - cnt_gt).astype(jnp.float32) * tval
        neg_total = jnp.sum(neg)

        n = acc_ref[2]
        ll_ref[0, 0] = acc_ref[0] / n
        lc_ref[0, 0] = (acc_ref[1] + neg_total) / n


def kernel(loc_data, conf_data, priors, targets):
    B, P, C = conf_data.shape
    pad = _PP - P
    pri_cm = jnp.pad(priors.T, ((0, 0), (0, pad)),
                     constant_values=10.0).reshape(4, _R, _LANES)

    nsteps = B // _IPS
    last = nsteps - 1
    ll, lc = pl.pallas_call(
        functools.partial(_mb_kernel, P, B),
        grid=(nsteps + 1,),
        in_specs=[
            pl.BlockSpec((_IPS, _O, 5),
                         lambda b: (jnp.minimum(b, last), 0, 0),
                         memory_space=pltpu.SMEM),
            pl.BlockSpec((_IPS, P, C),
                         lambda b: (jnp.minimum(b, last), 0, 0)),
            pl.BlockSpec((_IPS, P, 4),
                         lambda b: (jnp.minimum(b, last), 0, 0)),
            pl.BlockSpec((4, _R, _LANES), lambda b: (0, 0, 0)),
        ],
        out_specs=[
            pl.BlockSpec((1, 1), lambda b: (0, 0), memory_space=pltpu.SMEM),
            pl.BlockSpec((1, 1), lambda b: (0, 0), memory_space=pltpu.SMEM),
        ],
        out_shape=[jax.ShapeDtypeStruct((1, 1), jnp.float32)] * 2,
        scratch_shapes=[
            pltpu.SMEM((3,), jnp.float32),
            pltpu.VMEM((B, _R, _LANES), jnp.int32),
            pltpu.VMEM((B, 1), jnp.int32),
            pltpu.VMEM((_IPS * _C, _R, _LANES), jnp.float32),
            pltpu.VMEM((_IPS * 4, _R, _LANES), jnp.float32),
        ],
        compiler_params=pltpu.CompilerParams(
            dimension_semantics=("arbitrary",)),
    )(targets, conf_data, loc_data, pri_cm)
    return ll[0, 0], lc[0, 0]


# speedup vs baseline: 1.7424x; 1.7424x over previous
import functools

import jax
import jax.numpy as jnp
from jax import lax
from jax.experimental import pallas as pl
from jax.experimental.pallas import tpu as pltpu

_C = 21
_O = 16
_LANES = 128
_R = 69
_PP = _R * _LANES
_IPS = 2


def _rmax(x):
    return jnp.max(jnp.max(x, axis=0, keepdims=True), axis=1, keepdims=True)


def _rmin(x):
    return jnp.min(jnp.min(x, axis=0, keepdims=True), axis=1, keepdims=True)


def _rsum(x):
    return jnp.sum(jnp.sum(x, axis=0, keepdims=True), axis=1, keepdims=True)


def _rsum12(x):
    return jnp.sum(jnp.sum(x, axis=2, keepdims=True), axis=1, keepdims=True)


def _rmax12(x):
    return jnp.max(jnp.max(x, axis=2, keepdims=True), axis=1, keepdims=True)


def _smooth_l1(x):
    ax = jnp.abs(x)
    return jnp.where(ax < 1.0, 0.5 * ax * ax, ax - 0.5)


def _one_image(P, targ_ref, conf_ref, loc_ref, pri_ref, j):
    lin = (lax.broadcasted_iota(jnp.int32, (_R, _LANES), 0) * _LANES
           + lax.broadcasted_iota(jnp.int32, (_R, _LANES), 1))
    valid = lin < P

    cx = pri_ref[0]
    cy = pri_ref[1]
    pw = pri_ref[2]
    ph = pri_ref[3]
    px1 = cx - pw / 2.0
    py1 = cy - ph / 2.0
    px2 = cx + pw / 2.0
    py2 = cy + ph / 2.0
    area_p = (px2 - px1) * (py2 - py1)

    tx1 = [targ_ref[j, o, 0] for o in range(_O)]
    ty1 = [targ_ref[j, o, 1] for o in range(_O)]
    tx2 = [targ_ref[j, o, 2] for o in range(_O)]
    ty2 = [targ_ref[j, o, 3] for o in range(_O)]
    tlab = [targ_ref[j, o, 4] for o in range(_O)]

    bt_over = jnp.full((_R, _LANES), -1.0, jnp.float32)
    bt_idx = jnp.zeros((_R, _LANES), jnp.int32)
    bp = []
    for o in range(_O):
        iw = jnp.clip(jnp.minimum(tx2[o], px2) - jnp.maximum(tx1[o], px1),
                      0.0, None)
        ih = jnp.clip(jnp.minimum(ty2[o], py2) - jnp.maximum(ty1[o], py1),
                      0.0, None)
        inter = iw * ih
        area_t = (tx2[o] - tx1[o]) * (ty2[o] - ty1[o])
        iou = inter / (area_t + area_p - inter + 1e-10)
        upd = iou > bt_over
        bt_idx = jnp.where(upd, o, bt_idx)
        bt_over = jnp.where(upd, iou, bt_over)
        m = _rmax(iou)
        bp.append(_rmin(jnp.where(iou == m, lin, jnp.int32(2 ** 30))))
    for o in range(_O):
        mo = lin == bp[o]
        bt_over = jnp.where(mo, 2.0, bt_over)
        bt_idx = jnp.where(mo, o, bt_idx)

    mx1 = jnp.zeros_like(cx)
    my1 = jnp.zeros_like(cx)
    mx2 = jnp.zeros_like(cx)
    my2 = jnp.zeros_like(cx)
    lab = jnp.zeros_like(cx)
    for o in range(_O):
        sel = bt_idx == o
        mx1 = jnp.where(sel, tx1[o], mx1)
        my1 = jnp.where(sel, ty1[o], my1)
        mx2 = jnp.where(sel, tx2[o], mx2)
        my2 = jnp.where(sel, ty2[o], my2)
        lab = jnp.where(sel, tlab[o], lab)
    conf_t = jnp.where(bt_over < 0.5, 0.0, lab + 1.0)
    pos = conf_t > 0.0

    ecx = ((mx1 + mx2) / 2.0 - cx) / (0.1 * pw)
    ecy = ((my1 + my2) / 2.0 - cy) / (0.1 * ph)
    ew = jnp.log(jnp.clip((mx2 - mx1) / pw, 1e-10, None)) / 0.2
    eh = jnp.log(jnp.clip((my2 - my1) / ph, 1e-10, None)) / 0.2

    lx = loc_ref[4 * j + 0]
    ly = loc_ref[4 * j + 1]
    lw = loc_ref[4 * j + 2]
    lh = loc_ref[4 * j + 3]
    dcx = cx + lx * 0.1 * pw
    dcy = cy + ly * 0.1 * ph
    dw = pw * jnp.exp(lw * 0.2)
    dh = ph * jnp.exp(lh * 0.2)

    sl = (_smooth_l1((dcx - dw / 2.0) - ecx)
          + _smooth_l1((dcy - dh / 2.0) - ecy)
          + _smooth_l1((dcx + dw / 2.0) - ew)
          + _smooth_l1((dcy + dh / 2.0) - eh))
    loss_l_img = _rsum(jnp.where(pos, sl, 0.0))

    m0 = conf_ref[_C * j + 0]
    for c in range(1, _C):
        m0 = jnp.maximum(m0, conf_ref[_C * j + c])
    c0 = conf_ref[_C * j + 0]
    s = jnp.exp(c0 - m0)
    g = c0
    for c in range(1, _C):
        cc = conf_ref[_C * j + c]
        s = s + jnp.exp(cc - m0)
        g = jnp.where(conf_t == c, cc, g)
    lse = jnp.log(s) + m0
    ce = lse - g
    lca = jnp.where(jnp.logical_and(valid, jnp.logical_not(pos)), ce, 0.0)

    npos = _rsum(jnp.where(pos, 1, 0))
    pos_ce = _rsum(jnp.where(pos, ce, 0.0))
    return loss_l_img, pos_ce, npos, lax.bitcast_convert_type(lca, jnp.int32)


def _mb_kernel(P, B, targ_ref, conf_ref, loc_ref, pri_ref, ll_ref, lc_ref,
               acc_ref, bits_ref, npk_ref):
    b = pl.program_id(0)
    nsteps = B // _IPS

    @pl.when(b == 0)
    def _init():
        acc_ref[0] = 0.0
        acc_ref[1] = 0.0
        acc_ref[2] = 0.0

    @pl.when(b < nsteps)
    def _per_step():
        for j in range(_IPS):
            loss_l_img, pos_ce, npos, bits = _one_image(
                P, targ_ref, conf_ref, loc_ref, pri_ref, j)
            img = b * _IPS + j
            bits_ref[img] = bits
            npk_ref[pl.ds(img, 1)] = npos
            acc_ref[0] = acc_ref[0] + loss_l_img[0, 0]
            acc_ref[1] = acc_ref[1] + pos_ce[0, 0]
            acc_ref[2] = acc_ref[2] + npos[0, 0].astype(jnp.float32)

    @pl.when(b == nsteps)
    def _mine():
        bits3 = bits_ref[...]
        lca3 = lax.bitcast_convert_type(bits3, jnp.float32)
        k3 = jnp.minimum(3 * npk_ref[...], P - 1).reshape(B, 1, 1)

        def bs_body(_, carry):
            lo, hi = carry
            q = jnp.maximum((hi - lo) // 4, 1)
            m1 = lo + q
            m2 = lo + 2 * q
            m3 = lo + 3 * q
            c1 = _rsum12(jnp.where(bits3 >= m1, 1, 0)) >= k3
            c2 = _rsum12(jnp.where(bits3 >= m2, 1, 0)) >= k3
            c3 = _rsum12(jnp.where(bits3 >= m3, 1, 0)) >= k3
            nlo = jnp.where(c3, m3, jnp.where(c2, m2, jnp.where(c1, m1, lo)))
            nhi = jnp.where(jnp.logical_not(c1), m1,
                            jnp.where(jnp.logical_not(c2), m2,
                                      jnp.where(jnp.logical_not(c3), m3, hi)))
            return nlo, nhi

        lo0 = jnp.zeros((B, 1, 1), jnp.int32)
        hi0 = jnp.full((B, 1, 1), 0x7F800000, jnp.int32)
        lo, _ = lax.fori_loop(0, 17, bs_body, (lo0, hi0))

        gt = bits3 > lo
        cnt_gt = _rsum12(jnp.where(gt, 1, 0))
        sum_gt = _rsum12(jnp.where(gt, lca3, 0.0))
        tval = _rmax12(jnp.where(bits3 <= lo, lca3, 0.0))
        neg = sum_gt + (k3 - cnt_gt).astype(jnp.float32) * tval
        neg_total = jnp.sum(neg)

        n = acc_ref[2]
        ll_ref[0, 0] = acc_ref[0] / n
        lc_ref[0, 0] = (acc_ref[1] + neg_total) / n


def kernel(loc_data, conf_data, priors, targets):
    B, P, C = conf_data.shape
    pad = _PP - P
    conf_cm = jnp.moveaxis(conf_data, 2, 1)
    conf_cm = jnp.pad(conf_cm, ((0, 0), (0, 0), (0, pad)))
    conf_cm = conf_cm.reshape(B * C, _R, _LANES)
    loc_cm = jnp.moveaxis(loc_data, 2, 1)
    loc_cm = jnp.pad(loc_cm, ((0, 0), (0, 0), (0, pad)))
    loc_cm = loc_cm.reshape(B * 4, _R, _LANES)
    pri_cm = jnp.pad(priors.T, ((0, 0), (0, pad)),
                     constant_values=10.0).reshape(4, _R, _LANES)

    nsteps = B // _IPS
    last = nsteps - 1
    ll, lc = pl.pallas_call(
        functools.partial(_mb_kernel, P, B),
        grid=(nsteps + 1,),
        in_specs=[
            pl.BlockSpec((_IPS, _O, 5),
                         lambda b: (jnp.minimum(b, last), 0, 0),
                         memory_space=pltpu.SMEM),
            pl.BlockSpec((_IPS * _C, _R, _LANES),
                         lambda b: (jnp.minimum(b, last), 0, 0)),
            pl.BlockSpec((_IPS * 4, _R, _LANES),
                         lambda b: (jnp.minimum(b, last), 0, 0)),
            pl.BlockSpec((4, _R, _LANES), lambda b: (0, 0, 0)),
        ],
        out_specs=[
            pl.BlockSpec((1, 1), lambda b: (0, 0), memory_space=pltpu.SMEM),
            pl.BlockSpec((1, 1), lambda b: (0, 0), memory_space=pltpu.SMEM),
        ],
        out_shape=[jax.ShapeDtypeStruct((1, 1), jnp.float32)] * 2,
        scratch_shapes=[
            pltpu.SMEM((3,), jnp.float32),
            pltpu.VMEM((B, _R, _LANES), jnp.int32),
            pltpu.VMEM((B, 1), jnp.int32),
        ],
        compiler_params=pltpu.CompilerParams(
            dimension_semantics=("arbitrary",)),
    )(targets, conf_cm, loc_cm, pri_cm)
    return ll[0, 0], lc[0, 0]
